# Initial kernel scaffold; baseline (speedup 1.0000x reference)
import jax
import jax.numpy as jnp
from jax.experimental import pallas as pl


def kernel(x, edge_index, edge_attr, batch, W1_0, b1_0, W2_0, b2_0, W1_1, b1_1, W2_1, b2_1, W1_2, b1_2, W2_2, b2_2, lin_W, lin_b):
    raise NotImplementedError("write your pallas kernel here")


# speedup vs baseline: 5.5395x; 5.5395x over previous
import functools

import jax
import jax.numpy as jnp
from jax import lax
from jax.experimental import pallas as pl
from jax.experimental.pallas import tpu as pltpu
from jax.experimental.pallas import tpu_sc as plsc

NC = 2
NS = 16
NW = NC * NS
CHUNK = 128

B_SEG = 64


def _make_sc_agg(n_rows, d, nchunk):
    rpt = ((-(-(n_rows + 1) // NS)) + 7) // 8 * 8
    npad = rpt * NS
    full, rem = divmod(rpt, CHUNK)

    mesh = plsc.VectorSubcoreMesh(
        core_axis_name="c", subcore_axis_name="s",
        num_cores=NC, num_subcores=NS)

    @functools.partial(
        pl.kernel,
        out_type=jax.ShapeDtypeStruct((NC, npad, d), jnp.float32),
        mesh=mesh,
        scratch_types=[
            pltpu.VMEM((nchunk, CHUNK), jnp.int32),
            pltpu.VMEM((nchunk, CHUNK), jnp.int32),
            pltpu.VMEM((CHUNK, d), jnp.float32),
            pltpu.VMEM_SHARED((npad, d), jnp.float32),
            pltpu.SemaphoreType.DMA,
        ],
        compiler_params=pltpu.CompilerParams(use_tc_tiling_on_sc=False),
    )
    def sc_agg(h_hbm, src_hbm, dst_hbm, out_hbm, idx_s, idx_d, rows, acc, sem):
        cid = lax.axis_index("c")
        sid = lax.axis_index("s")
        wid = sid * NC + cid

        def zero_body(i, _):
            r = i // (d // 16)
            col = (i % (d // 16)) * 16
            rows[r, pl.ds(col, 16)] = jnp.zeros((16,), jnp.float32)
            return 0
        lax.fori_loop(0, CHUNK * d // 16, zero_body, 0)
        zbase = sid * rpt
        for k in range(full):
            pltpu.sync_copy(rows, acc.at[pl.ds(zbase + k * CHUNK, CHUNK)])
        if rem:
            pltpu.sync_copy(rows.at[pl.ds(0, rem)],
                            acc.at[pl.ds(zbase + full * CHUNK, rem)])
        plsc.subcore_barrier()

        pltpu.sync_copy(src_hbm.at[wid], idx_s)
        pltpu.sync_copy(dst_hbm.at[wid], idx_d)

        def chunk_body(ci, _):
            pltpu.async_copy(h_hbm.at[idx_s.at[ci]], rows, sem).wait()
            pltpu.sync_copy(rows, acc.at[idx_d.at[ci]], add=True)
            return 0
        lax.fori_loop(0, nchunk, chunk_body, 0)
        plsc.subcore_barrier()

        pltpu.sync_copy(acc.at[pl.ds(zbase, rpt)],
                        out_hbm.at[cid, pl.ds(zbase, rpt)])

    return sc_agg, npad


def _pick_bm(n):
    for bm in (512, 400, 256, 200, 128, 80, 40, 16, 8):
        if n % bm == 0:
            return bm
    return n


def _mlp_body(h_ref, a0_ref, a1_ref, w1_ref, b1_ref, w2_ref, b2_ref, o_ref):
    hs = h_ref[...] + a0_ref[...] + a1_ref[...]
    z = jnp.dot(hs, w1_ref[...], preferred_element_type=jnp.float32) + b1_ref[...]
    z = jnp.maximum(z, 0.0)
    z = jnp.dot(z, w2_ref[...], preferred_element_type=jnp.float32) + b2_ref[...]
    o_ref[...] = jnp.maximum(z, 0.0)


def _mlp_layer(h, a0, a1, w1, b1, w2, b2):
    n, din = h.shape
    hdim = w1.shape[1]
    bm = _pick_bm(n)
    return pl.pallas_call(
        _mlp_body,
        grid=(n // bm,),
        in_specs=[
            pl.BlockSpec((bm, din), lambda i: (i, 0)),
            pl.BlockSpec((bm, din), lambda i: (i, 0)),
            pl.BlockSpec((bm, din), lambda i: (i, 0)),
            pl.BlockSpec((din, hdim), lambda i: (0, 0)),
            pl.BlockSpec((1, hdim), lambda i: (0, 0)),
            pl.BlockSpec((hdim, hdim), lambda i: (0, 0)),
            pl.BlockSpec((1, hdim), lambda i: (0, 0)),
        ],
        out_specs=pl.BlockSpec((bm, hdim), lambda i: (i, 0)),
        out_shape=jax.ShapeDtypeStruct((n, hdim), jnp.float32),
    )(h, a0, a1, w1, b1.reshape(1, -1), w2, b2.reshape(1, -1))


def _final_layer(h, a0, a1, w1, b1, w2, b2, lin_w, lin_b, batch_i32):
    n, din = h.shape
    hdim = w1.shape[1]
    bm = _pick_bm(n)

    def body(h_ref, a0_ref, a1_ref, w1_ref, b1_ref, w2_ref, b2_ref,
             lw_ref, lb_ref, bt_ref, o_ref):
        hs = h_ref[...] + a0_ref[...] + a1_ref[...]
        z = jnp.dot(hs, w1_ref[...], preferred_element_type=jnp.float32) + b1_ref[...]
        z = jnp.maximum(z, 0.0)
        z = jnp.dot(z, w2_ref[...], preferred_element_type=jnp.float32) + b2_ref[...]
        hh = jnp.maximum(z, 0.0)
        y = jnp.dot(hh, lw_ref[...], preferred_element_type=jnp.float32)
        seg = lax.broadcasted_iota(jnp.int32, (bm, B_SEG), 1)
        oh = (bt_ref[...] == seg).astype(jnp.float32)
        contrib = jnp.sum(oh * y, axis=0, keepdims=True)

        @pl.when(pl.program_id(0) == 0)
        def _():
            o_ref[...] = contrib + lb_ref[...]

        @pl.when(pl.program_id(0) != 0)
        def _():
            o_ref[...] = o_ref[...] + contrib

    out = pl.pallas_call(
        body,
        grid=(n // bm,),
        in_specs=[
            pl.BlockSpec((bm, din), lambda i: (i, 0)),
            pl.BlockSpec((bm, din), lambda i: (i, 0)),
            pl.BlockSpec((bm, din), lambda i: (i, 0)),
            pl.BlockSpec((din, hdim), lambda i: (0, 0)),
            pl.BlockSpec((1, hdim), lambda i: (0, 0)),
            pl.BlockSpec((hdim, hdim), lambda i: (0, 0)),
            pl.BlockSpec((1, hdim), lambda i: (0, 0)),
            pl.BlockSpec((hdim, 1), lambda i: (0, 0)),
            pl.BlockSpec((1, 1), lambda i: (0, 0)),
            pl.BlockSpec((bm, 1), lambda i: (i, 0)),
        ],
        out_specs=pl.BlockSpec((1, B_SEG), lambda i: (0, 0)),
        out_shape=jax.ShapeDtypeStruct((1, B_SEG), jnp.float32),
    )(h, a0, a1, w1, b1.reshape(1, -1), w2, b2.reshape(1, -1),
      lin_w, lin_b.reshape(1, 1), batch_i32)
    return out[0]


def kernel(x, edge_index, edge_attr, batch,
           W1_0, b1_0, W2_0, b2_0, W1_1, b1_1, W2_1, b2_1,
           W1_2, b1_2, W2_2, b2_2, lin_W, lin_b):
    n, d = x.shape
    e = edge_index.shape[1]
    nchunk = -(-e // (NW * CHUNK))
    e_pad = NW * nchunk * CHUNK

    src = edge_index[0]
    dst = edge_index[1]
    if e_pad > e:
        src = jnp.concatenate([src, jnp.zeros((e_pad - e,), jnp.int32)])
        dst = jnp.concatenate([dst, jnp.full((e_pad - e,), n, jnp.int32)])
    src = src.reshape(NW, nchunk, CHUNK)
    dst = dst.reshape(NW, nchunk, CHUNK)

    batch_i32 = batch.astype(jnp.int32).reshape(n, 1)

    layers = [(W1_0, b1_0, W2_0, b2_0), (W1_1, b1_1, W2_1, b2_1),
              (W1_2, b1_2, W2_2, b2_2)]

    h = x
    for li, (w1, bb1, w2, bb2) in enumerate(layers):
        sc_agg, npad = _make_sc_agg(n, h.shape[1], nchunk)
        agg = sc_agg(h, src, dst)
        a0 = agg[0, :n]
        a1 = agg[1, :n]
        if li < 2:
            h = _mlp_layer(h, a0, a1, w1, bb1, w2, bb2)
        else:
            out = _final_layer(h, a0, a1, w1, bb1, w2, bb2, lin_W, lin_b,
                               batch_i32)
    return out
